# baseline (speedup 1.0000x reference)
import functools

import jax
import jax.numpy as jnp
from jax import lax
from jax.experimental import pallas as pl
from jax.experimental.pallas import tpu as pltpu
from jax.experimental.pallas import tpu_sc as plsc

_NC = 2
_NS = 16
_CH = 128


def kernel(user_idx, year, num_ratings, user_table, year_table,
           rating_table, year_bounds, rating_bounds):
    B = user_idx.shape[0]
    E = user_table.shape[1]
    EP = 16
    user_table = jnp.pad(user_table, ((0, 0), (0, EP - E)))
    year_table = jnp.pad(year_table, ((0, 0), (0, EP - E)))
    rating_table = jnp.pad(rating_table, ((0, 0), (0, EP - E)))
    nbnd = year_bounds.shape[0]
    nbins = year_table.shape[0]
    NW = _NC * _NS
    bpw = B // NW
    nch = bpw // _CH
    mesh = plsc.VectorSubcoreMesh(core_axis_name="c", subcore_axis_name="s")

    @functools.partial(
        pl.kernel,
        mesh=mesh,
        out_type=jax.ShapeDtypeStruct((3 * B, EP), jnp.float32),
        compiler_params=pltpu.CompilerParams(use_tc_tiling_on_sc=False),
        scratch_types=[
            pltpu.VMEM((nch, _CH), jnp.int32),
            pltpu.VMEM((nch, _CH), jnp.int32),
            pltpu.VMEM((nch, _CH), jnp.int32),
            pltpu.VMEM((nch, _CH), jnp.int32),
            pltpu.VMEM((nch, _CH), jnp.int32),
            pltpu.VMEM((nch, _CH), jnp.int32),
            pltpu.VMEM((nch, _CH), jnp.float32),
            pltpu.VMEM((nch, _CH), jnp.float32),
            pltpu.VMEM((32,), jnp.float32),
            pltpu.VMEM((32,), jnp.float32),
            pltpu.VMEM((nch, _CH, EP), jnp.float32),
            pltpu.VMEM((nch, _CH, EP), jnp.float32),
            pltpu.VMEM((nch, _CH, EP), jnp.float32),
            pltpu.SemaphoreType.DMA,
        ],
    )
    def sc_kernel(uidx_h, year_h, rate_h, utab_h, ytab_h, rtab_h,
                  ybnd_h, rbnd_h, out_h,
                  idx_v, uoi_v, yoi_v, roi_v, ybin_v, rbin_v, yv_v, rv_v,
                  ybnd_v, rbnd_v, ublk, yblk, rblk, sem):
        wid = lax.axis_index("s") * _NC + lax.axis_index("c")
        base = wid * bpw

        for j in range(nch):
            pltpu.sync_copy(uidx_h.at[pl.ds(base + j * _CH, _CH)], idx_v.at[j])
            pltpu.sync_copy(year_h.at[pl.ds(base + j * _CH, _CH)], yv_v.at[j])
            pltpu.sync_copy(rate_h.at[pl.ds(base + j * _CH, _CH)], rv_v.at[j])
        pltpu.sync_copy(ybnd_h, ybnd_v.at[pl.ds(0, nbnd)])
        pltpu.sync_copy(rbnd_h, rbnd_v.at[pl.ds(0, nbnd)])

        ucps = [pltpu.async_copy(utab_h.at[idx_v.at[j]], ublk.at[j], sem)
                for j in range(nch)]

        lane = lax.iota(jnp.int32, 16)
        yb0 = ybnd_v[pl.ds(0, 16)]
        yb1 = ybnd_v[pl.ds(16, 16)]
        rb0 = rbnd_v[pl.ds(0, 16)]
        rb1 = rbnd_v[pl.ds(16, 16)]

        gdn = lax.GatherDimensionNumbers(
            offset_dims=(), collapsed_slice_dims=(0,), start_index_map=(0,))

        def bcast(vec, idx):
            return lax.gather(vec, idx.reshape(16, 1), gdn, (1,),
                              mode=lax.GatherScatterMode.PROMISE_IN_BOUNDS)

        def rank(b0, b1, v):
            pos = jnp.zeros((16,), jnp.int32)
            for sz in (16, 8, 4, 2, 1):
                nxt = pos + sz
                probe = jnp.minimum(nxt - 1, nbnd - 1)
                g0 = bcast(b0, jnp.minimum(probe, 15))
                g1 = bcast(b1, jnp.clip(probe - 16, 0, 15))
                bv = jnp.where(probe < 16, g0, g1)
                take = (nxt <= nbnd) & (bv <= v)
                pos = jnp.where(take, nxt, pos)
            return jnp.minimum(pos, nbins - 1)

        def chunk(j, i8):
            off = i8 * 16
            yv = yv_v[j, pl.ds(off, 16)]
            rv = rv_v[j, pl.ds(off, 16)]
            ybin_v[j, pl.ds(off, 16)] = rank(yb0, yb1, yv)
            rbin_v[j, pl.ds(off, 16)] = rank(rb0, rb1, rv)
            g0 = (base + j * _CH) * 3 + i8 * 48
            uo = g0 + lane * 3
            uoi_v[j, pl.ds(off, 16)] = uo
            yoi_v[j, pl.ds(off, 16)] = uo + 1
            roi_v[j, pl.ds(off, 16)] = uo + 2

        for j in range(nch):
            def body(i8, carry, j=j):
                chunk(j, i8)
                return carry
            lax.fori_loop(0, _CH // 16, body, 0)

        ycps = [pltpu.async_copy(ytab_h.at[ybin_v.at[j]], yblk.at[j], sem)
                for j in range(nch)]
        rcps = [pltpu.async_copy(rtab_h.at[rbin_v.at[j]], rblk.at[j], sem)
                for j in range(nch)]
        for c in ucps + ycps + rcps:
            c.wait()

        scps = []
        for j in range(nch):
            scps.append(pltpu.async_copy(ublk.at[j], out_h.at[uoi_v.at[j]], sem))
            scps.append(pltpu.async_copy(yblk.at[j], out_h.at[yoi_v.at[j]], sem))
            scps.append(pltpu.async_copy(rblk.at[j], out_h.at[roi_v.at[j]], sem))
        for c in scps:
            c.wait()

    out = sc_kernel(user_idx, year, num_ratings, user_table, year_table,
                    rating_table, year_bounds, rating_bounds)
    return out[:, :E].reshape(B, 3 * E)
